# baseline (speedup 1.0000x reference)
import functools

import jax
import jax.numpy as jnp
from jax import lax
from jax.experimental import pallas as pl
from jax.experimental.pallas import tpu as pltpu
from jax.experimental.pallas import tpu_sc as plsc

BATCH = 16384
FEAT_DIM = 64
LANES = 16

_info = plsc.get_sparse_core_info()
NUM_CORES = _info.num_cores
NUM_SUBCORES = _info.num_subcores
NUM_WORKERS = NUM_CORES * NUM_SUBCORES
B_PER_W = BATCH // NUM_WORKERS
GROUPS = B_PER_W // LANES


def _sc_body(feat_hbm, y_hbm, table_hbm, out_hbm, idx_v, rows_v, feat_v,
             out_v, sem):
    wid = lax.axis_index("s") * NUM_CORES + lax.axis_index("c")
    base = wid * B_PER_W

    pltpu.sync_copy(y_hbm.at[pl.ds(base, B_PER_W)], idx_v)
    gather = pltpu.async_copy(table_hbm.at[idx_v], rows_v, sem)
    pltpu.sync_copy(feat_hbm.at[pl.ds(base, B_PER_W)], feat_v)
    gather.wait()

    lane = lax.iota(jnp.int32, LANES)

    def group_body(g, carry):
        outvec = jnp.zeros((LANES,), jnp.float32)
        for j in range(LANES):
            r = g * LANES + j
            acc = jnp.zeros((LANES,), jnp.float32)
            for q in range(FEAT_DIM // LANES):
                f = feat_v[r, pl.ds(q * LANES, LANES)]
                w = rows_v[r, pl.ds(q * LANES, LANES)]
                acc = acc + f * w
            total = jnp.sum(acc)
            outvec = jnp.where(lane == j, total, outvec)
        out_v[pl.ds(g * LANES, LANES)] = outvec
        return carry

    lax.fori_loop(0, GROUPS, group_body, 0)

    pltpu.sync_copy(out_v, out_hbm.at[pl.ds(base, B_PER_W)])


@jax.jit
def _projection_head(feat, y, embed_weight):
    mesh = plsc.VectorSubcoreMesh(core_axis_name="c", subcore_axis_name="s")
    kern = functools.partial(
        pl.kernel,
        out_type=jax.ShapeDtypeStruct((BATCH,), jnp.float32),
        mesh=mesh,
        scratch_types=[
            pltpu.VMEM((B_PER_W,), jnp.int32),
            pltpu.VMEM((B_PER_W, FEAT_DIM), jnp.float32),
            pltpu.VMEM((B_PER_W, FEAT_DIM), jnp.float32),
            pltpu.VMEM((B_PER_W,), jnp.float32),
            pltpu.SemaphoreType.DMA,
        ],
        compiler_params=pltpu.CompilerParams(
            needs_layout_passes=False, use_tc_tiling_on_sc=False
        ),
    )(_sc_body)
    return kern(feat, y, embed_weight)


def kernel(feat, y, embed_weight):
    return _projection_head(feat, y.astype(jnp.int32), embed_weight)
